# baseline (speedup 1.0000x reference)
import jax
import jax.numpy as jnp
from jax import lax
from jax.experimental import pallas as pl
from jax.experimental.pallas import tpu as pltpu

_NUM_E = 1024
_DIM = 64
_TN = 1024
_CCOST = 0.25
_EPS = 1e-10
_PREC = None


def _body(x_ref, e_ref, qst_ref, idx_ref, cnt_ref, loss_ref, perp_ref):
    i = pl.program_id(0)
    nsteps = pl.num_programs(0)
    xt = x_ref[...]
    emb = e_ref[...]

    sx = jnp.sum(xt * xt, axis=1, keepdims=True)
    ee = emb * emb
    ones = jnp.ones((1, _DIM), jnp.float32)
    se_row = lax.dot_general(ones, ee, (((1,), (1,)), ((), ())),
                             precision=lax.Precision.HIGHEST,
                             preferred_element_type=jnp.float32)
    mm = lax.dot_general(xt, emb, (((1,), (1,)), ((), ())),
                         precision=_PREC,
                         preferred_element_type=jnp.float32)
    d = (sx + se_row) - 2.0 * mm

    dmin = jnp.min(d, axis=1, keepdims=True)
    codes = lax.broadcasted_iota(jnp.int32, (_TN, _NUM_E), 1)
    idx = jnp.min(jnp.where(d == dmin, codes, _NUM_E), axis=1,
                  keepdims=True)
    idx_ref[...] = idx

    onehot = (codes == idx).astype(jnp.float32)
    q = lax.dot_general(onehot, emb, (((1,), (0,)), ((), ())),
                        precision=_PREC,
                        preferred_element_type=jnp.float32)
    qst_ref[...] = xt + (q - xt)

    part_sse = jnp.sum((q - xt) ** 2)
    part_cnt = jnp.sum(onehot, axis=0, keepdims=True)

    @pl.when(i == 0)
    def _init():
        cnt_ref[...] = jnp.zeros_like(cnt_ref)
        loss_ref[...] = jnp.zeros_like(loss_ref)
        perp_ref[...] = jnp.zeros_like(perp_ref)

    cnt_ref[...] += part_cnt
    loss_ref[...] += part_sse

    @pl.when(i == nsteps - 1)
    def _finish():
        n_tok = nsteps * _TN
        m = loss_ref[...] / (n_tok * _DIM)
        loss_ref[...] = m + _CCOST * m
        avg = cnt_ref[...] / n_tok
        ent = jnp.sum(avg * jnp.log(avg + _EPS))
        perp_ref[...] = jnp.exp(-ent) * jnp.ones_like(perp_ref)


def kernel(x, embedding_weight):
    B, C, H, W = x.shape
    N = B * H * W
    xf = jnp.transpose(x, (0, 2, 3, 1)).reshape(N, C)
    grid = (N // _TN,)
    qst, idx, _cnt, loss, perp = pl.pallas_call(
        _body,
        grid=grid,
        in_specs=[
            pl.BlockSpec((_TN, C), lambda i: (i, 0)),
            pl.BlockSpec((_NUM_E, C), lambda i: (0, 0)),
        ],
        out_specs=[
            pl.BlockSpec((_TN, C), lambda i: (i, 0)),
            pl.BlockSpec((_TN, 1), lambda i: (i, 0)),
            pl.BlockSpec((1, _NUM_E), lambda i: (0, 0)),
            pl.BlockSpec((1, 1), lambda i: (0, 0)),
            pl.BlockSpec((1, 1), lambda i: (0, 0)),
        ],
        out_shape=[
            jax.ShapeDtypeStruct((N, C), jnp.float32),
            jax.ShapeDtypeStruct((N, 1), jnp.int32),
            jax.ShapeDtypeStruct((1, _NUM_E), jnp.float32),
            jax.ShapeDtypeStruct((1, 1), jnp.float32),
            jax.ShapeDtypeStruct((1, 1), jnp.float32),
        ],
        compiler_params=pltpu.CompilerParams(
            dimension_semantics=("arbitrary",)),
    )(xf, embedding_weight)
    quantized_st = jnp.transpose(qst.reshape(B, H, W, C), (0, 3, 1, 2))
    return (quantized_st, loss.reshape(()), perp.reshape(()),
            idx.reshape(B, H, W))
